# baseline (speedup 1.0000x reference)
import jax
import jax.numpy as jnp
from jax import lax
from jax.experimental import pallas as pl
from jax.experimental.pallas import tpu as pltpu
from jax.experimental.pallas import tpu_sc as plsc

D = 768
L = 16
NC, NS = 2, 16
NW = NC * NS
B_TOTAL = 4 * 2048
P_W = B_TOTAL // NW
CHUNK = 16
N_CHUNKS = P_W // CHUNK
DEPTH = 4


def _emb_body(idx_hbm, table_hbm, out_hbm,
              idx_int, idx0_v, idx1_v, a_bufs, b_bufs, sa, sb, sw):
    wid = lax.axis_index("s") * NC + lax.axis_index("c")
    base = wid * P_W
    pltpu.sync_copy(idx_hbm.at[pl.ds(2 * base, 2 * P_W)], idx_int)
    lanes2 = lax.iota(jnp.int32, L) * 2
    for k in range(P_W // L):
        sl = pl.ds(k * L, L)
        idx0_v[sl] = plsc.load_gather(idx_int, [lanes2 + (2 * L * k)])
        idx1_v[sl] = plsc.load_gather(idx_int, [lanes2 + (2 * L * k + 1)])

    def gathers(c, s):
        sl = pl.ds(c * CHUNK, CHUNK)
        return (
            pltpu.async_copy(table_hbm.at[idx0_v.at[sl]], a_bufs[s], sa[s]),
            pltpu.async_copy(table_hbm.at[idx1_v.at[sl]], b_bufs[s], sb[s]),
        )

    ga = [None] * DEPTH
    wb = [None] * DEPTH
    for c in range(DEPTH - 1):
        ga[c] = gathers(c, c)
    for c in range(N_CHUNKS):
        s = c % DEPTH
        ga[s][0].wait()
        ga[s][1].wait()
        a_v, b_v = a_bufs[s], b_bufs[s]

        def add_rows(i, _):
            for r in range(2):
                for j in range(D // L):
                    sl = pl.ds(j * L, L)
                    plsc.addupdate(a_v.at[2 * i + r, sl], b_v[2 * i + r, sl])
            return 0

        lax.fori_loop(0, CHUNK // 2, add_rows, 0)
        wb[s] = pltpu.async_copy(
            a_v, out_hbm.at[pl.ds(base + c * CHUNK, CHUNK)], sw[s])
        nxt = c + DEPTH - 1
        if nxt < N_CHUNKS:
            s2 = nxt % DEPTH
            if wb[s2] is not None:
                wb[s2].wait()
            ga[s2] = gathers(nxt, s2)
    for d in wb:
        if d is not None:
            d.wait()


@jax.jit
def kernel(inputs, we):
    idx = inputs.reshape(-1).astype(jnp.int32)
    mesh = plsc.VectorSubcoreMesh(core_axis_name="c", subcore_axis_name="s")
    run = pl.kernel(
        _emb_body,
        out_type=jax.ShapeDtypeStruct((B_TOTAL, D), jnp.float32),
        mesh=mesh,
        compiler_params=pltpu.CompilerParams(needs_layout_passes=False),
        scratch_types=[
            pltpu.VMEM((2 * P_W,), jnp.int32),
            pltpu.VMEM((P_W,), jnp.int32),
            pltpu.VMEM((P_W,), jnp.int32),
            [pltpu.VMEM((CHUNK, D), jnp.float32) for _ in range(DEPTH)],
            [pltpu.VMEM((CHUNK, D), jnp.float32) for _ in range(DEPTH)],
            [pltpu.SemaphoreType.DMA for _ in range(DEPTH)],
            [pltpu.SemaphoreType.DMA for _ in range(DEPTH)],
            [pltpu.SemaphoreType.DMA for _ in range(DEPTH)],
        ],
    )
    out = run(idx, we)
    return out.reshape(inputs.shape[0], inputs.shape[1], D)
